# baseline (speedup 1.0000x reference)
import functools

import jax
import jax.numpy as jnp
from jax import lax
from jax.experimental import pallas as pl
from jax.experimental.pallas import tpu as pltpu
from jax.experimental.pallas import tpu_sc as plsc

_B = 16384
_D = 128
_NC = 2
_NS = 16
_NW = _NC * _NS
_BPW = _B // _NW

_GCH = [(0, 256), (256, 256)]
_WCH = [(0, 256), (256, 256)]
_UNROLL = 2


def _rsqrt16(x):
    i = lax.bitcast_convert_type(x, jnp.int32)
    i = jnp.full((16,), 0x5F3759DF, jnp.int32) - lax.shift_right_arithmetic(
        i, jnp.full((16,), 1, jnp.int32))
    y = lax.bitcast_convert_type(i, jnp.float32)
    for _ in range(2):
        y = y * (1.5 - 0.5 * x * y * y)
    return y


def _body(nodes_hbm, table_hbm, out_hbm, idx_v, rows_v, gsems, osem):
    wid = lax.axis_index("s") * _NC + lax.axis_index("c")
    base = wid * _BPW
    gathers = []
    for i, (g0, gn) in enumerate(_GCH):
        pltpu.sync_copy(
            nodes_hbm.at[pl.ds(base + g0, gn)], idx_v.at[pl.ds(g0, gn)]
        )
        gathers.append(
            pltpu.async_copy(
                table_hbm.at[idx_v.at[pl.ds(g0, gn)]],
                rows_v.at[pl.ds(g0, gn)],
                gsems[i],
            )
        )
    outs = []
    for w0, wn in _WCH:
        need = w0 + wn
        for i, (g0, gn) in enumerate(_GCH):
            if gathers[i] is not None and g0 < need and g0 + gn > w0:
                gathers[i].wait()
                gathers[i] = None

        @plsc.parallel_loop(w0, w0 + wn, step=1, unroll=_UNROLL)
        def _row(r):
            vs = [rows_v[r, pl.ds(j * 16, 16)] for j in range(_D // 16)]
            a0 = vs[0] * vs[0]
            a1 = vs[1] * vs[1]
            for j in range(2, _D // 16, 2):
                a0 = vs[j] * vs[j] + a0
                a1 = vs[j + 1] * vs[j + 1] + a1
            ss = jnp.full((16,), 0, jnp.float32) + jnp.sum(a0 + a1)
            inv = _rsqrt16(jnp.maximum(ss, 1e-24))
            for j, v in enumerate(vs):
                rows_v[r, pl.ds(j * 16, 16)] = v * inv

        outs.append(
            pltpu.async_copy(
                rows_v.at[pl.ds(w0, wn)],
                out_hbm.at[pl.ds(base + w0, wn)],
                osem,
            )
        )
    for o in outs:
        o.wait()


@jax.jit
def kernel(nodes, emb_table):
    mesh = plsc.VectorSubcoreMesh(core_axis_name="c", subcore_axis_name="s")
    run = functools.partial(
        pl.kernel,
        out_type=jax.ShapeDtypeStruct((_B, _D), jnp.float32),
        mesh=mesh,
        compiler_params=pltpu.CompilerParams(
            needs_layout_passes=False, disable_bounds_checks=True
        ),
        scratch_types=[
            pltpu.VMEM((_BPW,), jnp.int32),
            pltpu.VMEM((_BPW, _D), jnp.float32),
            [pltpu.SemaphoreType.DMA] * len(_GCH),
            pltpu.SemaphoreType.DMA,
        ],
    )(_body)
    return run(nodes, emb_table)


# speedup vs baseline: 1.0148x; 1.0148x over previous
import functools

import jax
import jax.numpy as jnp
from jax import lax
from jax.experimental import pallas as pl
from jax.experimental.pallas import tpu as pltpu
from jax.experimental.pallas import tpu_sc as plsc

_B = 16384
_D = 128
_NC = 2
_NS = 16
_NW = _NC * _NS
_BPW = _B // _NW

_GCH = [(0, 256), (256, 256)]
_WCH = [(0, 256), (256, 128), (384, 128)]
_UNROLL = 2


def _rsqrt16(x):
    i = lax.bitcast_convert_type(x, jnp.int32)
    i = jnp.full((16,), 0x5F3759DF, jnp.int32) - lax.shift_right_arithmetic(
        i, jnp.full((16,), 1, jnp.int32))
    y = lax.bitcast_convert_type(i, jnp.float32)
    for _ in range(2):
        y = y * (1.5 - 0.5 * x * y * y)
    return y


def _body(nodes_hbm, table_hbm, out_hbm, idx_v, rows_v, gsems, osem):
    wid = lax.axis_index("s") * _NC + lax.axis_index("c")
    base = wid * _BPW
    gathers = []
    for i, (g0, gn) in enumerate(_GCH):
        pltpu.sync_copy(
            nodes_hbm.at[pl.ds(base + g0, gn)], idx_v.at[pl.ds(g0, gn)]
        )
        gathers.append(
            pltpu.async_copy(
                table_hbm.at[idx_v.at[pl.ds(g0, gn)]],
                rows_v.at[pl.ds(g0, gn)],
                gsems[i],
            )
        )
    outs = []
    for w0, wn in _WCH:
        need = w0 + wn
        for i, (g0, gn) in enumerate(_GCH):
            if gathers[i] is not None and g0 < need and g0 + gn > w0:
                gathers[i].wait()
                gathers[i] = None

        @plsc.parallel_loop(w0, w0 + wn, step=1, unroll=_UNROLL)
        def _row(r):
            vs = [rows_v[r, pl.ds(j * 16, 16)] for j in range(_D // 16)]
            a0 = vs[0] * vs[0]
            a1 = vs[1] * vs[1]
            for j in range(2, _D // 16, 2):
                a0 = vs[j] * vs[j] + a0
                a1 = vs[j + 1] * vs[j + 1] + a1
            ss = jnp.full((16,), 0, jnp.float32) + jnp.sum(a0 + a1)
            inv = _rsqrt16(jnp.maximum(ss, 1e-24))
            for j, v in enumerate(vs):
                rows_v[r, pl.ds(j * 16, 16)] = v * inv

        outs.append(
            pltpu.async_copy(
                rows_v.at[pl.ds(w0, wn)],
                out_hbm.at[pl.ds(base + w0, wn)],
                osem,
            )
        )
    for o in outs:
        o.wait()


@jax.jit
def kernel(nodes, emb_table):
    mesh = plsc.VectorSubcoreMesh(core_axis_name="c", subcore_axis_name="s")
    run = functools.partial(
        pl.kernel,
        out_type=jax.ShapeDtypeStruct((_B, _D), jnp.float32),
        mesh=mesh,
        compiler_params=pltpu.CompilerParams(needs_layout_passes=False),
        scratch_types=[
            pltpu.VMEM((_BPW,), jnp.int32),
            pltpu.VMEM((_BPW, _D), jnp.float32),
            [pltpu.SemaphoreType.DMA] * len(_GCH),
            pltpu.SemaphoreType.DMA,
        ],
    )(_body)
    return run(nodes, emb_table)
